# initial kernel scaffold (unmeasured)
import functools

import jax
import jax.numpy as jnp
from jax import lax
from jax.experimental import pallas as pl
from jax.experimental.pallas import tpu as pltpu

N_DEV = 16


def kernel(x, assign, W1, W2):
    T, D = x.shape
    E, _, F = W1.shape

    def body(x_ref, a_ref, w1_ref, w2_ref, out_ref,
             buf_x, buf_a, buf_acc, fin_acc,
             sx, sa, sc, rx, ra, rc, sfin, rfin, credit):
        my = lax.axis_index("i")
        left = lax.rem(my - 1 + N_DEV, N_DEV)
        right = lax.rem(my + 1, N_DEV)

        barrier = pltpu.get_barrier_semaphore()
        for nbr in (left, right):
            pl.semaphore_signal(barrier, inc=1, device_id=(nbr,),
                                device_id_type=pl.DeviceIdType.MESH)
        pl.semaphore_wait(barrier, 2)

        buf_x[0] = x_ref[...]
        buf_a[0] = a_ref[...]
        buf_acc[0] = jnp.zeros((T, D), jnp.float32)

        for h in range(N_DEV):
            s = h % 2
            r = (h + 1) % 2

            a = buf_a[s]
            xs = buf_x[s]
            acc = buf_acc[s]
            for e in range(E):
                mask = a == (my * E + e)
                xm = jnp.where(mask, xs, 0.0)
                h1 = jnp.maximum(
                    jnp.dot(xm, w1_ref[e], preferred_element_type=jnp.float32),
                    0.0)
                acc = acc + jnp.dot(h1, w2_ref[e],
                                    preferred_element_type=jnp.float32)
            buf_acc[s] = acc

            if h < N_DEV - 1:
                if h >= 1:
                    pl.semaphore_wait(credit, 1)
                copies = [
                    pltpu.make_async_remote_copy(
                        src_ref=buf_x.at[s], dst_ref=buf_x.at[r],
                        send_sem=sx.at[s], recv_sem=rx.at[r],
                        device_id=(right,),
                        device_id_type=pl.DeviceIdType.MESH),
                    pltpu.make_async_remote_copy(
                        src_ref=buf_a.at[s], dst_ref=buf_a.at[r],
                        send_sem=sa.at[s], recv_sem=ra.at[r],
                        device_id=(right,),
                        device_id_type=pl.DeviceIdType.MESH),
                    pltpu.make_async_remote_copy(
                        src_ref=buf_acc.at[s], dst_ref=buf_acc.at[r],
                        send_sem=sc.at[s], recv_sem=rc.at[r],
                        device_id=(right,),
                        device_id_type=pl.DeviceIdType.MESH),
                ]
                for c in copies:
                    c.start()
                for c in copies:
                    c.wait()
                pl.semaphore_signal(credit, inc=1, device_id=(left,),
                                    device_id_type=pl.DeviceIdType.MESH)
            else:
                fin = pltpu.make_async_remote_copy(
                    src_ref=buf_acc.at[s], dst_ref=fin_acc,
                    send_sem=sfin, recv_sem=rfin,
                    device_id=(right,),
                    device_id_type=pl.DeviceIdType.MESH)
                fin.start()
                fin.wait()
                out_ref[...] = fin_acc[...]

    a2d = assign.reshape(T, 1)

    return pl.pallas_call(
        body,
        out_shape=jax.ShapeDtypeStruct((T, D), jnp.float32),
        in_specs=[pl.BlockSpec(memory_space=pltpu.VMEM)] * 4,
        out_specs=pl.BlockSpec(memory_space=pltpu.VMEM),
        scratch_shapes=[
            pltpu.VMEM((2, T, D), jnp.float32),
            pltpu.VMEM((2, T, 1), jnp.int32),
            pltpu.VMEM((2, T, D), jnp.float32),
            pltpu.VMEM((T, D), jnp.float32),
            pltpu.SemaphoreType.DMA((2,)),
            pltpu.SemaphoreType.DMA((2,)),
            pltpu.SemaphoreType.DMA((2,)),
            pltpu.SemaphoreType.DMA((2,)),
            pltpu.SemaphoreType.DMA((2,)),
            pltpu.SemaphoreType.DMA((2,)),
            pltpu.SemaphoreType.DMA,
            pltpu.SemaphoreType.DMA,
            pltpu.SemaphoreType.REGULAR,
        ],
        compiler_params=pltpu.CompilerParams(collective_id=0),
    )(x, a2d, W1, W2)


# baseline (device time: 483250 ns/iter reference)
import functools

import jax
import jax.numpy as jnp
from jax import lax
from jax.experimental import pallas as pl
from jax.experimental.pallas import tpu as pltpu

N_DEV = 16


def kernel(x, assign, W1, W2):
    T, D = x.shape
    E, _, F = W1.shape

    def body(x_ref, a_ref, w1_ref, w2_ref, out_ref,
             buf_x, buf_a, buf_acc, fin_acc,
             sx, sa, sc, rx, ra, rc, sfin, rfin, credit):
        my = lax.axis_index("i")
        left = lax.rem(my - 1 + N_DEV, N_DEV)
        right = lax.rem(my + 1, N_DEV)

        barrier = pltpu.get_barrier_semaphore()
        for nbr in (left, right):
            pl.semaphore_signal(barrier, inc=1, device_id=(nbr,),
                                device_id_type=pl.DeviceIdType.MESH)
        pl.semaphore_wait(barrier, 2)

        buf_x[0] = x_ref[...]
        buf_a[0] = a_ref[...]
        buf_acc[0] = jnp.zeros((T, D), jnp.float32)

        for h in range(N_DEV):
            s = h % 2
            r = (h + 1) % 2

            a = buf_a[s]
            xs = buf_x[s]
            acc = buf_acc[s]
            for e in range(E):
                mask = a == (my * E + e)
                xm = jnp.where(mask, xs, 0.0)
                h1 = jnp.maximum(
                    jnp.dot(xm, w1_ref[e], preferred_element_type=jnp.float32),
                    0.0)
                acc = acc + jnp.dot(h1, w2_ref[e],
                                    preferred_element_type=jnp.float32)
            buf_acc[s] = acc

            if h < N_DEV - 1:
                if h >= 1:
                    pl.semaphore_wait(credit, 1)
                copies = [
                    pltpu.make_async_remote_copy(
                        src_ref=buf_x.at[s], dst_ref=buf_x.at[r],
                        send_sem=sx.at[s], recv_sem=rx.at[r],
                        device_id=(right,),
                        device_id_type=pl.DeviceIdType.MESH),
                    pltpu.make_async_remote_copy(
                        src_ref=buf_a.at[s], dst_ref=buf_a.at[r],
                        send_sem=sa.at[s], recv_sem=ra.at[r],
                        device_id=(right,),
                        device_id_type=pl.DeviceIdType.MESH),
                    pltpu.make_async_remote_copy(
                        src_ref=buf_acc.at[s], dst_ref=buf_acc.at[r],
                        send_sem=sc.at[s], recv_sem=rc.at[r],
                        device_id=(right,),
                        device_id_type=pl.DeviceIdType.MESH),
                ]
                for c in copies:
                    c.start()
                for c in copies:
                    c.wait()
                if h < N_DEV - 2:
                    pl.semaphore_signal(credit, inc=1, device_id=(left,),
                                        device_id_type=pl.DeviceIdType.MESH)
            else:
                fin = pltpu.make_async_remote_copy(
                    src_ref=buf_acc.at[s], dst_ref=fin_acc,
                    send_sem=sfin, recv_sem=rfin,
                    device_id=(right,),
                    device_id_type=pl.DeviceIdType.MESH)
                fin.start()
                fin.wait()
                out_ref[...] = fin_acc[...]

    a2d = assign.reshape(T, 1)

    return pl.pallas_call(
        body,
        out_shape=jax.ShapeDtypeStruct((T, D), jnp.float32),
        in_specs=[pl.BlockSpec(memory_space=pltpu.VMEM)] * 4,
        out_specs=pl.BlockSpec(memory_space=pltpu.VMEM),
        scratch_shapes=[
            pltpu.VMEM((2, T, D), jnp.float32),
            pltpu.VMEM((2, T, 1), jnp.int32),
            pltpu.VMEM((2, T, D), jnp.float32),
            pltpu.VMEM((T, D), jnp.float32),
            pltpu.SemaphoreType.DMA((2,)),
            pltpu.SemaphoreType.DMA((2,)),
            pltpu.SemaphoreType.DMA((2,)),
            pltpu.SemaphoreType.DMA((2,)),
            pltpu.SemaphoreType.DMA((2,)),
            pltpu.SemaphoreType.DMA((2,)),
            pltpu.SemaphoreType.DMA,
            pltpu.SemaphoreType.DMA,
            pltpu.SemaphoreType.REGULAR,
        ],
        compiler_params=pltpu.CompilerParams(collective_id=0),
    )(x, a2d, W1, W2)


# device time: 156806 ns/iter; 3.0818x vs baseline; 3.0818x over previous
import jax
import jax.numpy as jnp
from jax import lax
from jax.experimental import pallas as pl
from jax.experimental.pallas import tpu as pltpu

N_DEV = 16
MESH = pl.DeviceIdType.MESH


def kernel(x, assign, W1, W2):
    T, D = x.shape
    E, _, F = W1.shape
    H = T // 2

    def body(x_ref, a_ref, w1_ref, w2_ref, out_ref,
             w1b, w2b,
             bxR, baR, bcR, finR, bxL, baL, bcL, finL,
             semsR, semsL, sfinR, rfinR, sfinL, rfinL,
             creditR, creditL):
        my = lax.axis_index("i")
        left = lax.rem(my - 1 + N_DEV, N_DEV)
        right = lax.rem(my + 1, N_DEV)

        barrier = pltpu.get_barrier_semaphore()
        for nbr in (left, right):
            pl.semaphore_signal(barrier, inc=1, device_id=(nbr,),
                                device_id_type=MESH)
        pl.semaphore_wait(barrier, 2)

        w1b[...] = w1_ref[...].astype(jnp.bfloat16)
        w2b[...] = w2_ref[...].astype(jnp.bfloat16)

        bxR[0] = x_ref[:H, :].astype(jnp.bfloat16)
        baR[0] = a_ref[:H, :]
        bcR[0] = jnp.zeros((H, D), jnp.bfloat16)
        bxL[0] = x_ref[H:, :].astype(jnp.bfloat16)
        baL[0] = a_ref[H:, :]
        bcL[0] = jnp.zeros((H, D), jnp.bfloat16)

        def contrib(xs, a):
            acc = None
            for e in range(E):
                mask = a == (my * E + e)
                xm = jnp.where(mask, xs, jnp.zeros_like(xs))
                h1 = jnp.maximum(
                    jnp.dot(xm, w1b[e], preferred_element_type=jnp.float32),
                    0.0).astype(jnp.bfloat16)
                c = jnp.dot(h1, w2b[e], preferred_element_type=jnp.float32)
                acc = c if acc is None else acc + c
            return acc.astype(jnp.bfloat16)

        def mk(buf, sems, kind, s_src, s_dst, dev):
            return pltpu.make_async_remote_copy(
                src_ref=buf.at[s_src], dst_ref=buf.at[s_dst],
                send_sem=sems.at[kind, s_src], recv_sem=sems.at[kind + 3, s_dst],
                device_id=(dev,), device_id_type=MESH)

        rings = (
            (bxR, baR, bcR, finR, semsR, sfinR, rfinR, creditR, right, left),
            (bxL, baL, bcL, finL, semsL, sfinL, rfinL, creditL, left, right),
        )

        for h in range(N_DEV):
            s = h % 2
            r = (h + 1) % 2

            for bx, ba, bc, fin, sems, sfin, rfin, credit, dn, up in rings:
                if h > 0:
                    mk(bx, sems, 0, r, s, dn).wait_recv()
                    mk(ba, sems, 1, r, s, dn).wait_recv()
                if h < N_DEV - 1:
                    if h >= 1:
                        pl.semaphore_wait(credit, 1)
                    mk(bx, sems, 0, s, r, dn).start()
                    mk(ba, sems, 1, s, r, dn).start()

            cR = contrib(bxR[s], baR[s])
            cL = contrib(bxL[s], baL[s])

            for (bx, ba, bc, fin, sems, sfin, rfin, credit, dn, up), c in (
                    (rings[0], cR), (rings[1], cL)):
                if h > 0:
                    mk(bc, sems, 2, r, s, dn).wait_recv()
                bc[s] = bc[s] + c
                if h < N_DEV - 1:
                    mk(bc, sems, 2, s, r, dn).start()
                else:
                    fd = pltpu.make_async_remote_copy(
                        src_ref=bc.at[s], dst_ref=fin,
                        send_sem=sfin, recv_sem=rfin,
                        device_id=(dn,), device_id_type=MESH)
                    fd.start()

            for bx, ba, bc, fin, sems, sfin, rfin, credit, dn, up in rings:
                if h < N_DEV - 1:
                    mk(bx, sems, 0, s, r, dn).wait_send()
                    mk(ba, sems, 1, s, r, dn).wait_send()
                    mk(bc, sems, 2, s, r, dn).wait_send()
                    if h < N_DEV - 2:
                        pl.semaphore_signal(credit, inc=1, device_id=(up,),
                                            device_id_type=MESH)

        for i, (bx, ba, bc, fin, sems, sfin, rfin, credit, dn, up) in (
                enumerate(rings)):
            fd = pltpu.make_async_remote_copy(
                src_ref=bc.at[(N_DEV - 1) % 2], dst_ref=fin,
                send_sem=sfin, recv_sem=rfin,
                device_id=(dn,), device_id_type=MESH)
            fd.wait_send()
            fd.wait_recv()
            if i == 0:
                out_ref[:H, :] = fin[...].astype(jnp.float32)
            else:
                out_ref[H:, :] = fin[...].astype(jnp.float32)

    a2d = assign.reshape(T, 1)

    return pl.pallas_call(
        body,
        out_shape=jax.ShapeDtypeStruct((T, D), jnp.float32),
        in_specs=[pl.BlockSpec(memory_space=pltpu.VMEM)] * 4,
        out_specs=pl.BlockSpec(memory_space=pltpu.VMEM),
        scratch_shapes=[
            pltpu.VMEM((E, D, F), jnp.bfloat16),
            pltpu.VMEM((E, F, D), jnp.bfloat16),
            pltpu.VMEM((2, H, D), jnp.bfloat16),
            pltpu.VMEM((2, H, 1), jnp.int32),
            pltpu.VMEM((2, H, D), jnp.bfloat16),
            pltpu.VMEM((H, D), jnp.bfloat16),
            pltpu.VMEM((2, H, D), jnp.bfloat16),
            pltpu.VMEM((2, H, 1), jnp.int32),
            pltpu.VMEM((2, H, D), jnp.bfloat16),
            pltpu.VMEM((H, D), jnp.bfloat16),
            pltpu.SemaphoreType.DMA((6, 2)),
            pltpu.SemaphoreType.DMA((6, 2)),
            pltpu.SemaphoreType.DMA,
            pltpu.SemaphoreType.DMA,
            pltpu.SemaphoreType.DMA,
            pltpu.SemaphoreType.DMA,
            pltpu.SemaphoreType.REGULAR,
            pltpu.SemaphoreType.REGULAR,
        ],
        compiler_params=pltpu.CompilerParams(collective_id=0),
    )(x, a2d, W1, W2)


# device time: 128331 ns/iter; 3.7657x vs baseline; 1.2219x over previous
import jax
import jax.numpy as jnp
from jax import lax
from jax.experimental import pallas as pl
from jax.experimental.pallas import tpu as pltpu

N_DEV = 16
K = 3
MESH = pl.DeviceIdType.MESH


def kernel(x, assign, W1, W2):
    T, D = x.shape
    E, _, F = W1.shape
    H = T // 2

    def body(x_ref, a_ref, w1_ref, w2_ref, out_ref,
             w1b, w2b,
             bxR, baR, bcR, finR, bxL, baL, bcL, finL,
             semsR, semsL, sfinR, rfinR, sfinL, rfinL,
             creditR, creditL):
        my = lax.axis_index("i")
        left = lax.rem(my - 1 + N_DEV, N_DEV)
        right = lax.rem(my + 1, N_DEV)

        barrier = pltpu.get_barrier_semaphore()
        for nbr in (left, right):
            pl.semaphore_signal(barrier, inc=1, device_id=(nbr,),
                                device_id_type=MESH)
        pl.semaphore_wait(barrier, 2)

        w1b[...] = w1_ref[...].astype(jnp.bfloat16)
        w2b[...] = w2_ref[...].astype(jnp.bfloat16)

        bxR[0] = x_ref[:H, :].astype(jnp.bfloat16)
        baR[0] = a_ref[:H, :]
        bcR[0] = jnp.zeros((H, D), jnp.bfloat16)
        bxL[0] = x_ref[H:, :].astype(jnp.bfloat16)
        baL[0] = a_ref[H:, :]
        bcL[0] = jnp.zeros((H, D), jnp.bfloat16)

        def contrib(xs, a):
            acc = None
            for e in range(E):
                mask = a == (my * E + e)
                xm = jnp.where(mask, xs, jnp.zeros_like(xs))
                h1 = jnp.maximum(
                    jnp.dot(xm, w1b[e], preferred_element_type=jnp.float32),
                    0.0).astype(jnp.bfloat16)
                c = jnp.dot(h1, w2b[e], preferred_element_type=jnp.float32)
                acc = c if acc is None else acc + c
            return acc.astype(jnp.bfloat16)

        def mk(buf, sems, kind, s_src, s_dst, dev):
            return pltpu.make_async_remote_copy(
                src_ref=buf.at[s_src], dst_ref=buf.at[s_dst],
                send_sem=sems.at[kind, s_src], recv_sem=sems.at[kind + 3, s_dst],
                device_id=(dev,), device_id_type=MESH)

        rings = (
            (bxR, baR, bcR, finR, semsR, sfinR, rfinR, creditR, right, left),
            (bxL, baL, bcL, finL, semsL, sfinL, rfinL, creditL, left, right),
        )

        for h in range(N_DEV):
            s = h % K
            r = (h + 1) % K
            p = (h - 1) % K

            for bx, ba, bc, fin, sems, sfin, rfin, credit, dn, up in rings:
                if h > 0:
                    mk(bx, sems, 0, r, s, dn).wait_recv()
                    mk(ba, sems, 1, r, s, dn).wait_recv()
                if 2 <= h <= N_DEV - 2:
                    pl.semaphore_wait(credit, 1)
                if h <= N_DEV - 2:
                    mk(bx, sems, 0, s, r, dn).start()
                    mk(ba, sems, 1, s, r, dn).start()

            cR = contrib(bxR[s], baR[s])
            cL = contrib(bxL[s], baL[s])

            for bx, ba, bc, fin, sems, sfin, rfin, credit, dn, up in rings:
                if h >= 1:
                    mk(bx, sems, 0, p, s, dn).wait_send()
                    mk(ba, sems, 1, p, s, dn).wait_send()
                    mk(bc, sems, 2, p, s, dn).wait_send()
                    if h <= N_DEV - 3:
                        pl.semaphore_signal(credit, inc=1, device_id=(up,),
                                            device_id_type=MESH)

            for (bx, ba, bc, fin, sems, sfin, rfin, credit, dn, up), c in (
                    (rings[0], cR), (rings[1], cL)):
                if h > 0:
                    mk(bc, sems, 2, r, s, dn).wait_recv()
                bc[s] = bc[s] + c
                if h <= N_DEV - 2:
                    mk(bc, sems, 2, s, r, dn).start()
                else:
                    fd = pltpu.make_async_remote_copy(
                        src_ref=bc.at[s], dst_ref=fin,
                        send_sem=sfin, recv_sem=rfin,
                        device_id=(dn,), device_id_type=MESH)
                    fd.start()

        for i, (bx, ba, bc, fin, sems, sfin, rfin, credit, dn, up) in (
                enumerate(rings)):
            fd = pltpu.make_async_remote_copy(
                src_ref=bc.at[(N_DEV - 1) % K], dst_ref=fin,
                send_sem=sfin, recv_sem=rfin,
                device_id=(dn,), device_id_type=MESH)
            fd.wait_send()
            fd.wait_recv()
            if i == 0:
                out_ref[:H, :] = fin[...].astype(jnp.float32)
            else:
                out_ref[H:, :] = fin[...].astype(jnp.float32)

    a2d = assign.reshape(T, 1)

    return pl.pallas_call(
        body,
        out_shape=jax.ShapeDtypeStruct((T, D), jnp.float32),
        in_specs=[pl.BlockSpec(memory_space=pltpu.VMEM)] * 4,
        out_specs=pl.BlockSpec(memory_space=pltpu.VMEM),
        scratch_shapes=[
            pltpu.VMEM((E, D, F), jnp.bfloat16),
            pltpu.VMEM((E, F, D), jnp.bfloat16),
            pltpu.VMEM((K, H, D), jnp.bfloat16),
            pltpu.VMEM((K, H, 1), jnp.int32),
            pltpu.VMEM((K, H, D), jnp.bfloat16),
            pltpu.VMEM((H, D), jnp.bfloat16),
            pltpu.VMEM((K, H, D), jnp.bfloat16),
            pltpu.VMEM((K, H, 1), jnp.int32),
            pltpu.VMEM((K, H, D), jnp.bfloat16),
            pltpu.VMEM((H, D), jnp.bfloat16),
            pltpu.SemaphoreType.DMA((6, K)),
            pltpu.SemaphoreType.DMA((6, K)),
            pltpu.SemaphoreType.DMA,
            pltpu.SemaphoreType.DMA,
            pltpu.SemaphoreType.DMA,
            pltpu.SemaphoreType.DMA,
            pltpu.SemaphoreType.REGULAR,
            pltpu.SemaphoreType.REGULAR,
        ],
        compiler_params=pltpu.CompilerParams(collective_id=0),
    )(x, a2d, W1, W2)


# device time: 127919 ns/iter; 3.7778x vs baseline; 1.0032x over previous
import jax
import jax.numpy as jnp
from jax import lax
from jax.experimental import pallas as pl
from jax.experimental.pallas import tpu as pltpu

N_DEV = 16
K = 3
MESH = pl.DeviceIdType.MESH

_CYCLE = [0, 4, 8, 12, 13, 9, 5, 1, 2, 6, 10, 14, 15, 11, 7, 3]
_NEXT = [0] * N_DEV
_PREV = [0] * N_DEV
for _k, _c in enumerate(_CYCLE):
    _NEXT[_c] = _CYCLE[(_k + 1) % N_DEV]
    _PREV[_c] = _CYCLE[(_k - 1) % N_DEV]


def kernel(x, assign, W1, W2):
    T, D = x.shape
    E, _, F = W1.shape
    H = T // 2

    def body(nbr_ref, x_ref, a_ref, w1_ref, w2_ref, out_ref,
             w1b, w2b,
             bxR, baR, bcR, finR, bxL, baL, bcL, finL,
             semsR, semsL, sfinR, rfinR, sfinL, rfinL,
             creditR, creditL):
        my = lax.axis_index("i")
        right = nbr_ref[0]
        left = nbr_ref[1]

        barrier = pltpu.get_barrier_semaphore()
        for nbr in (left, right):
            pl.semaphore_signal(barrier, inc=1, device_id=(nbr,),
                                device_id_type=MESH)
        pl.semaphore_wait(barrier, 2)

        w1b[...] = w1_ref[...].astype(jnp.bfloat16)
        w2b[...] = w2_ref[...].astype(jnp.bfloat16)

        bxR[0] = x_ref[:H, :].astype(jnp.bfloat16)
        baR[0] = a_ref[:H, :]
        bcR[0] = jnp.zeros((H, D), jnp.bfloat16)
        bxL[0] = x_ref[H:, :].astype(jnp.bfloat16)
        baL[0] = a_ref[H:, :]
        bcL[0] = jnp.zeros((H, D), jnp.bfloat16)

        def contrib(xs, a):
            acc = None
            for e in range(E):
                mask = a == (my * E + e)
                xm = jnp.where(mask, xs, jnp.zeros_like(xs))
                h1 = jnp.maximum(
                    jnp.dot(xm, w1b[e], preferred_element_type=jnp.float32),
                    0.0).astype(jnp.bfloat16)
                c = jnp.dot(h1, w2b[e], preferred_element_type=jnp.float32)
                acc = c if acc is None else acc + c
            return acc.astype(jnp.bfloat16)

        def mk(buf, sems, kind, s_src, s_dst, dev):
            return pltpu.make_async_remote_copy(
                src_ref=buf.at[s_src], dst_ref=buf.at[s_dst],
                send_sem=sems.at[kind, s_src], recv_sem=sems.at[kind + 3, s_dst],
                device_id=(dev,), device_id_type=MESH)

        rings = (
            (bxR, baR, bcR, finR, semsR, sfinR, rfinR, creditR, right, left),
            (bxL, baL, bcL, finL, semsL, sfinL, rfinL, creditL, left, right),
        )

        for h in range(N_DEV):
            s = h % K
            r = (h + 1) % K
            p = (h - 1) % K

            for bx, ba, bc, fin, sems, sfin, rfin, credit, dn, up in rings:
                if h > 0:
                    mk(bx, sems, 0, r, s, dn).wait_recv()
                    mk(ba, sems, 1, r, s, dn).wait_recv()
                if 2 <= h <= N_DEV - 2:
                    pl.semaphore_wait(credit, 1)
                if h <= N_DEV - 2:
                    mk(bx, sems, 0, s, r, dn).start()
                    mk(ba, sems, 1, s, r, dn).start()

            cR = contrib(bxR[s], baR[s])
            cL = contrib(bxL[s], baL[s])

            for bx, ba, bc, fin, sems, sfin, rfin, credit, dn, up in rings:
                if h >= 1:
                    mk(bx, sems, 0, p, s, dn).wait_send()
                    mk(ba, sems, 1, p, s, dn).wait_send()
                    mk(bc, sems, 2, p, s, dn).wait_send()
                    if h <= N_DEV - 3:
                        pl.semaphore_signal(credit, inc=1, device_id=(up,),
                                            device_id_type=MESH)

            for (bx, ba, bc, fin, sems, sfin, rfin, credit, dn, up), c in (
                    (rings[0], cR), (rings[1], cL)):
                if h > 0:
                    mk(bc, sems, 2, r, s, dn).wait_recv()
                bc[s] = bc[s] + c
                if h <= N_DEV - 2:
                    mk(bc, sems, 2, s, r, dn).start()
                else:
                    fd = pltpu.make_async_remote_copy(
                        src_ref=bc.at[s], dst_ref=fin,
                        send_sem=sfin, recv_sem=rfin,
                        device_id=(dn,), device_id_type=MESH)
                    fd.start()

        for i, (bx, ba, bc, fin, sems, sfin, rfin, credit, dn, up) in (
                enumerate(rings)):
            fd = pltpu.make_async_remote_copy(
                src_ref=bc.at[(N_DEV - 1) % K], dst_ref=fin,
                send_sem=sfin, recv_sem=rfin,
                device_id=(dn,), device_id_type=MESH)
            fd.wait_send()
            fd.wait_recv()
            if i == 0:
                out_ref[:H, :] = fin[...].astype(jnp.float32)
            else:
                out_ref[H:, :] = fin[...].astype(jnp.float32)

    a2d = assign.reshape(T, 1)
    my_out = lax.axis_index("i")
    nbrs = jnp.stack([jnp.array(_NEXT, jnp.int32)[my_out],
                      jnp.array(_PREV, jnp.int32)[my_out]])

    return pl.pallas_call(
        body,
        out_shape=jax.ShapeDtypeStruct((T, D), jnp.float32),
        in_specs=[pl.BlockSpec(memory_space=pltpu.SMEM)]
        + [pl.BlockSpec(memory_space=pltpu.VMEM)] * 4,
        out_specs=pl.BlockSpec(memory_space=pltpu.VMEM),
        scratch_shapes=[
            pltpu.VMEM((E, D, F), jnp.bfloat16),
            pltpu.VMEM((E, F, D), jnp.bfloat16),
            pltpu.VMEM((K, H, D), jnp.bfloat16),
            pltpu.VMEM((K, H, 1), jnp.int32),
            pltpu.VMEM((K, H, D), jnp.bfloat16),
            pltpu.VMEM((H, D), jnp.bfloat16),
            pltpu.VMEM((K, H, D), jnp.bfloat16),
            pltpu.VMEM((K, H, 1), jnp.int32),
            pltpu.VMEM((K, H, D), jnp.bfloat16),
            pltpu.VMEM((H, D), jnp.bfloat16),
            pltpu.SemaphoreType.DMA((6, K)),
            pltpu.SemaphoreType.DMA((6, K)),
            pltpu.SemaphoreType.DMA,
            pltpu.SemaphoreType.DMA,
            pltpu.SemaphoreType.DMA,
            pltpu.SemaphoreType.DMA,
            pltpu.SemaphoreType.REGULAR,
            pltpu.SemaphoreType.REGULAR,
        ],
        compiler_params=pltpu.CompilerParams(collective_id=0),
    )(nbrs, x, a2d, W1, W2)


# device time: 45182 ns/iter; 10.6956x vs baseline; 2.8312x over previous
import jax
import jax.numpy as jnp
from jax import lax
from jax.experimental import pallas as pl
from jax.experimental.pallas import tpu as pltpu

N_DEV = 16
C = 32
MESH = pl.DeviceIdType.MESH


def kernel(x, assign, W1, W2):
    T, D = x.shape
    E, _, F = W1.shape
    R = N_DEV * C

    def body(x_ref, a_ref, w1_ref, w2_ref, out_ref,
             w1b, w2b, selbig, xcbig, xbuf, ybuf, ybig,
             xsend, xrecv, ysend, yrecv):
        my = lax.axis_index("i")

        barrier = pltpu.get_barrier_semaphore()
        for d in range(1, N_DEV):
            peer = lax.rem(my + d, N_DEV)
            pl.semaphore_signal(barrier, inc=1, device_id=(peer,),
                                device_id_type=MESH)
        pl.semaphore_wait(barrier, N_DEV - 1)

        w1b[...] = w1_ref[...].astype(jnp.bfloat16)
        w2b[...] = w2_ref[...].astype(jnp.bfloat16)

        a_row = a_ref[...]
        gids = lax.broadcasted_iota(jnp.int32, (2 * N_DEV, T), 0)
        onehot = (a_row == gids).astype(jnp.float32)
        ti = lax.broadcasted_iota(jnp.int32, (T, T), 0)
        tj = lax.broadcasted_iota(jnp.int32, (T, T), 1)
        ltri = (ti <= tj).astype(jnp.float32)
        cum = jnp.dot(onehot, ltri, preferred_element_type=jnp.float32)
        rank_row = jnp.sum(onehot * (cum - 1.0), axis=0, keepdims=True)

        rows_d = lax.broadcasted_iota(jnp.int32, (R, T), 0) // C
        rows_k = lax.broadcasted_iota(jnp.int32, (R, T), 0) % C
        for e in range(E):
            dst_expert = 2 * lax.rem(my + rows_d, N_DEV) + e
            sel = (a_row == dst_expert) & (rank_row == rows_k.astype(jnp.float32))
            selbig[e] = sel.astype(jnp.bfloat16)

        xb = x_ref[...].astype(jnp.bfloat16)
        for e in range(E):
            xcbig[e] = jnp.dot(selbig[e], xb,
                               preferred_element_type=jnp.float32
                               ).astype(jnp.bfloat16)

        xbuf[:, 0] = xcbig[:, 0:C, :]
        xd = []
        for d in range(1, N_DEV):
            r = pltpu.make_async_remote_copy(
                src_ref=xcbig.at[:, d * C:(d + 1) * C, :],
                dst_ref=xbuf.at[:, d],
                send_sem=xsend.at[d], recv_sem=xrecv.at[d],
                device_id=(lax.rem(my + d, N_DEV),), device_id_type=MESH)
            r.start()
            xd.append(r)
        for r in xd:
            r.wait_recv()

        for e in range(E):
            xall = xbuf[e].reshape(R, D)
            h1 = jnp.maximum(
                jnp.dot(xall, w1b[e], preferred_element_type=jnp.float32),
                0.0).astype(jnp.bfloat16)
            ybig[e] = jnp.dot(h1, w2b[e],
                              preferred_element_type=jnp.float32
                              ).astype(jnp.bfloat16)

        ybuf[:, 0] = ybig[:, 0:C, :]
        yd = []
        for d in range(1, N_DEV):
            r = pltpu.make_async_remote_copy(
                src_ref=ybig.at[:, d * C:(d + 1) * C, :],
                dst_ref=ybuf.at[:, d],
                send_sem=ysend.at[d], recv_sem=yrecv.at[d],
                device_id=(lax.rem(my - d + N_DEV, N_DEV),),
                device_id_type=MESH)
            r.start()
            yd.append(r)
        for r in yd:
            r.wait_recv()

        acc = None
        for e in range(E):
            yall = ybuf[e].reshape(R, D)
            part = lax.dot_general(
                selbig[e], yall, (((0,), (0,)), ((), ())),
                preferred_element_type=jnp.float32)
            acc = part if acc is None else acc + part
        out_ref[...] = acc

        for r in xd:
            r.wait_send()
        for r in yd:
            r.wait_send()

    a2d = assign.reshape(1, T)

    return pl.pallas_call(
        body,
        out_shape=jax.ShapeDtypeStruct((T, D), jnp.float32),
        in_specs=[pl.BlockSpec(memory_space=pltpu.VMEM)] * 4,
        out_specs=pl.BlockSpec(memory_space=pltpu.VMEM),
        scratch_shapes=[
            pltpu.VMEM((E, D, F), jnp.bfloat16),
            pltpu.VMEM((E, F, D), jnp.bfloat16),
            pltpu.VMEM((E, R, T), jnp.bfloat16),
            pltpu.VMEM((E, R, D), jnp.bfloat16),
            pltpu.VMEM((E, N_DEV, C, D), jnp.bfloat16),
            pltpu.VMEM((E, N_DEV, C, D), jnp.bfloat16),
            pltpu.VMEM((E, R, D), jnp.bfloat16),
            pltpu.SemaphoreType.DMA((N_DEV,)),
            pltpu.SemaphoreType.DMA((N_DEV,)),
            pltpu.SemaphoreType.DMA((N_DEV,)),
            pltpu.SemaphoreType.DMA((N_DEV,)),
        ],
        compiler_params=pltpu.CompilerParams(collective_id=0),
    )(x, a2d, W1, W2)
